# baseline (speedup 1.0000x reference)
import jax
import jax.numpy as jnp
from jax import lax
from jax.experimental import pallas as pl
from jax.experimental.pallas import tpu as pltpu


def _expert_ffn(xh, wg, wu, wd):
    g = lax.dot_general(xh, wg, (((1,), (1,)), ((), ())),
                        preferred_element_type=jnp.float32)
    u = lax.dot_general(xh, wu, (((1,), (1,)), ((), ())),
                        preferred_element_type=jnp.float32)
    h = (g * jax.nn.sigmoid(g) * u).astype(jnp.bfloat16)
    return lax.dot_general(h, wd, (((1,), (1,)), ((), ())),
                           preferred_element_type=jnp.float32)


def _moe_kernel(xb_ref, gw_ref, wg_ref, wu_ref, wd_ref,
                wsg_ref, wsu_ref, wsd_ref, out_ref, cmb_ref):
    e = pl.program_id(0)
    T, E = cmb_ref.shape
    H = T // 4

    @pl.when(e == 0)
    def _router_and_shared():
        xb = xb_ref[...]
        logits = lax.dot_general(xb, gw_ref[...].astype(jnp.bfloat16),
                                 (((1,), (1,)), ((), ())),
                                 preferred_element_type=jnp.float32)
        cols = lax.broadcasted_iota(jnp.int32, (T, E), 1)
        m1 = jnp.max(logits, axis=1, keepdims=True)
        i1 = jnp.min(jnp.where(logits == m1, cols, E), axis=1, keepdims=True)
        mask1 = cols == i1
        l2 = jnp.where(mask1, -jnp.inf, logits)
        m2 = jnp.max(l2, axis=1, keepdims=True)
        i2 = jnp.min(jnp.where(l2 == m2, cols, E), axis=1, keepdims=True)
        mask2 = cols == i2
        p1 = 1.0 / (1.0 + jnp.exp(m2 - m1))
        cmb_ref[...] = jnp.where(mask1, p1, 0.0) + jnp.where(mask2, 1.0 - p1, 0.0)
        for hh in range(4):
            rows = pl.ds(hh * H, H)
            out_ref[rows, :] = _expert_ffn(xb_ref[rows, :], wsg_ref[...],
                                           wsu_ref[...], wsd_ref[...])

    cmb = cmb_ref[...]
    cols = lax.broadcasted_iota(jnp.int32, (T, E), 1)
    wcol = jnp.sum(jnp.where(cols == e, cmb, 0.0), axis=1, keepdims=True)
    for hh in range(4):
        rows = pl.ds(hh * H, H)
        o = _expert_ffn(xb_ref[rows, :], wg_ref[0], wu_ref[0], wd_ref[0])
        out_ref[rows, :] = out_ref[rows, :] + o * wcol[hh * H:(hh + 1) * H, :]


def kernel(hidden_states, gate_w, w_gate, w_up, w_down, ws_gate, ws_up, ws_down):
    orig_shape = hidden_states.shape
    x = hidden_states.reshape(-1, orig_shape[-1])
    T, D = x.shape
    E, FF, _ = w_gate.shape
    SFF = ws_gate.shape[0]

    xb = x.astype(jnp.bfloat16)
    wg = w_gate.astype(jnp.bfloat16)
    wu = w_up.astype(jnp.bfloat16)
    wd = w_down.astype(jnp.bfloat16)
    wsg = ws_gate.astype(jnp.bfloat16)
    wsu = ws_up.astype(jnp.bfloat16)
    wsd = ws_down.astype(jnp.bfloat16)

    y = pl.pallas_call(
        _moe_kernel,
        grid=(E,),
        in_specs=[
            pl.BlockSpec((T, D), lambda e: (0, 0)),
            pl.BlockSpec((E, D), lambda e: (0, 0)),
            pl.BlockSpec((1, FF, D), lambda e: (e, 0, 0)),
            pl.BlockSpec((1, FF, D), lambda e: (e, 0, 0)),
            pl.BlockSpec((1, D, FF), lambda e: (e, 0, 0)),
            pl.BlockSpec((SFF, D), lambda e: (0, 0)),
            pl.BlockSpec((SFF, D), lambda e: (0, 0)),
            pl.BlockSpec((D, SFF), lambda e: (0, 0)),
        ],
        out_specs=pl.BlockSpec((T, D), lambda e: (0, 0)),
        out_shape=jax.ShapeDtypeStruct((T, D), jnp.float32),
        scratch_shapes=[pltpu.VMEM((T, E), jnp.float32)],
    )(xb, gate_w, wg, wu, wd, wsg, wsu, wsd)

    return y.reshape(orig_shape)


# speedup vs baseline: 1.0210x; 1.0210x over previous
import jax
import jax.numpy as jnp
from jax import lax
from jax.experimental import pallas as pl
from jax.experimental.pallas import tpu as pltpu


def _expert_ffn(xh, wg, wu, wd, scale=None):
    g = lax.dot_general(xh, wg, (((1,), (1,)), ((), ())),
                        preferred_element_type=jnp.float32)
    u = lax.dot_general(xh, wu, (((1,), (1,)), ((), ())),
                        preferred_element_type=jnp.float32)
    h = g * jax.nn.sigmoid(g) * u
    if scale is not None:
        h = h * scale
    return lax.dot_general(h.astype(jnp.bfloat16), wd, (((1,), (1,)), ((), ())),
                           preferred_element_type=jnp.float32)


def _moe_kernel(xb_ref, gw_ref, wg_ref, wu_ref, wd_ref,
                wsg_ref, wsu_ref, wsd_ref, out_ref, cmb_ref):
    e = pl.program_id(0)
    T, E = cmb_ref.shape
    H = T // 2

    @pl.when(e == 0)
    def _router_and_shared():
        xb = xb_ref[...]
        logits = lax.dot_general(xb, gw_ref[...].astype(jnp.bfloat16),
                                 (((1,), (1,)), ((), ())),
                                 preferred_element_type=jnp.float32)
        cols = lax.broadcasted_iota(jnp.int32, (T, E), 1)
        m1 = jnp.max(logits, axis=1, keepdims=True)
        i1 = jnp.min(jnp.where(logits == m1, cols, E), axis=1, keepdims=True)
        mask1 = cols == i1
        l2 = jnp.where(mask1, -jnp.inf, logits)
        m2 = jnp.max(l2, axis=1, keepdims=True)
        i2 = jnp.min(jnp.where(l2 == m2, cols, E), axis=1, keepdims=True)
        mask2 = cols == i2
        p1 = 1.0 / (1.0 + jnp.exp(m2 - m1))
        cmb_ref[...] = jnp.where(mask1, p1, 0.0) + jnp.where(mask2, 1.0 - p1, 0.0)
        for hh in range(2):
            rows = pl.ds(hh * H, H)
            out_ref[rows, :] = _expert_ffn(xb_ref[rows, :], wsg_ref[...],
                                           wsu_ref[...], wsd_ref[...])

    cmb = cmb_ref[...]
    cols = lax.broadcasted_iota(jnp.int32, (T, E), 1)
    wcol = jnp.sum(jnp.where(cols == e, cmb, 0.0), axis=1, keepdims=True)
    for hh in range(2):
        rows = pl.ds(hh * H, H)
        o = _expert_ffn(xb_ref[rows, :], wg_ref[0], wu_ref[0], wd_ref[0],
                        scale=wcol[hh * H:(hh + 1) * H, :])
        out_ref[rows, :] = out_ref[rows, :] + o


def kernel(hidden_states, gate_w, w_gate, w_up, w_down, ws_gate, ws_up, ws_down):
    orig_shape = hidden_states.shape
    x = hidden_states.reshape(-1, orig_shape[-1])
    T, D = x.shape
    E, FF, _ = w_gate.shape
    SFF = ws_gate.shape[0]

    xb = x.astype(jnp.bfloat16)
    wg = w_gate.astype(jnp.bfloat16)
    wu = w_up.astype(jnp.bfloat16)
    wd = w_down.astype(jnp.bfloat16)
    wsg = ws_gate.astype(jnp.bfloat16)
    wsu = ws_up.astype(jnp.bfloat16)
    wsd = ws_down.astype(jnp.bfloat16)

    y = pl.pallas_call(
        _moe_kernel,
        grid=(E,),
        in_specs=[
            pl.BlockSpec((T, D), lambda e: (0, 0)),
            pl.BlockSpec((E, D), lambda e: (0, 0)),
            pl.BlockSpec((1, FF, D), lambda e: (e, 0, 0)),
            pl.BlockSpec((1, FF, D), lambda e: (e, 0, 0)),
            pl.BlockSpec((1, D, FF), lambda e: (e, 0, 0)),
            pl.BlockSpec((SFF, D), lambda e: (0, 0)),
            pl.BlockSpec((SFF, D), lambda e: (0, 0)),
            pl.BlockSpec((D, SFF), lambda e: (0, 0)),
        ],
        out_specs=pl.BlockSpec((T, D), lambda e: (0, 0)),
        out_shape=jax.ShapeDtypeStruct((T, D), jnp.float32),
        scratch_shapes=[pltpu.VMEM((T, E), jnp.float32)],
    )(xb, gate_w, wg, wu, wd, wsg, wsu, wsd)

    return y.reshape(orig_shape)
